# Initial kernel scaffold; baseline (speedup 1.0000x reference)
import jax
import jax.numpy as jnp
from jax.experimental import pallas as pl


def kernel(x, edge_index, edge_weight, W1_l, b1, W1_r, W3_l, b3, W3_r):
    raise NotImplementedError("write your pallas kernel here")


# speedup vs baseline: 14.2570x; 14.2570x over previous
import functools

import jax
import jax.numpy as jnp
from jax import lax
from jax.experimental import pallas as pl
from jax.experimental.pallas import tpu as pltpu
from jax.experimental.pallas import tpu_sc as plsc

_LANES = 16
_IDXW = 128
_KROWS = 32
_NCORES = 2
_NSUB = 16
_NTILES = _NCORES * _NSUB
_TC_BLK = 5000


def _make_sc_pass(n_nodes, dh, rows_per_tile, with_count):
  assert rows_per_tile % _KROWS == 0
  n_chunks = rows_per_tile // _KROWS
  n_rows_total = _NTILES * rows_per_tile
  rows0 = (-(-n_nodes // _NSUB) + 7) // 8 * 8
  rows_last = n_nodes - (_NSUB - 1) * rows0
  assert rows_last > 0
  n_pad = n_nodes + 8

  mesh = plsc.VectorSubcoreMesh(core_axis_name="c", subcore_axis_name="s")

  out_type = [jax.ShapeDtypeStruct((_NCORES, n_nodes, dh), jnp.float32)]
  if with_count:
    out_type += [jax.ShapeDtypeStruct((n_nodes,), jnp.float32)] * 2

  scratch = [
      pltpu.VMEM((_KROWS, _IDXW), jnp.int32),
      pltpu.VMEM((_KROWS, _IDXW), jnp.int32),
      pltpu.VMEM((_KROWS * _IDXW,), jnp.float32),
      pltpu.VMEM((_IDXW, dh), jnp.float32),
      pltpu.VMEM_SHARED((n_pad, dh), jnp.float32),
      pltpu.SemaphoreType.DMA,
  ]
  if with_count:
    scratch += [
        pltpu.VMEM((_IDXW,), jnp.float32),
        pltpu.VMEM_SHARED((n_pad,), jnp.float32),
    ]

  def body(*refs):
    if with_count:
      (table, src_h, dst_h, ew_h, z8_h, z1_h, acc_o, cnt0_o, cnt1_o,
       srcb, dstb, ewb, rows, acc_s, gsem, onesv, cnt_s) = refs
    else:
      (table, src_h, dst_h, ew_h, z8_h, acc_o,
       srcb, dstb, ewb, rows, acc_s, gsem) = refs
      z1_h = onesv = cnt_s = None

    c = lax.axis_index("c")
    s = lax.axis_index("s")
    wid = c * _NSUB + s
    off0 = s * rows0

    def copy_span(get_src, get_dst):
      @pl.when(s < _NSUB - 1)
      def _():
        pltpu.sync_copy(get_src(off0, rows0), get_dst(off0, rows0))

      @pl.when(s == _NSUB - 1)
      def _():
        lo = (_NSUB - 1) * rows0
        pltpu.sync_copy(get_src(lo, rows_last), get_dst(lo, rows_last))

    copy_span(lambda o, r: z8_h.at[pl.ds(o, r)],
              lambda o, r: acc_s.at[pl.ds(o, r)])
    if with_count:
      copy_span(lambda o, r: z1_h.at[pl.ds(o, r)],
                lambda o, r: cnt_s.at[pl.ds(o, r)])
      for i in range(_IDXW // _LANES):
        onesv[pl.ds(i * _LANES, _LANES)] = jnp.full((_LANES,), 1.0, jnp.float32)

    plsc.subcore_barrier()

    iota = lax.iota(jnp.int32, _LANES)
    step = (iota >= (_LANES // 2)).astype(jnp.int32)
    col = jnp.bitwise_and(iota, dh - 1)
    n_vregs = _IDXW * dh // _LANES

    def chunk_body(gc, carry):
      base = wid * rows_per_tile + gc * _KROWS
      pltpu.sync_copy(src_h.at[pl.ds(base, _KROWS)], srcb)
      pltpu.sync_copy(dst_h.at[pl.ds(base, _KROWS)], dstb)
      pltpu.sync_copy(ew_h.at[pl.ds(base * _IDXW, _KROWS * _IDXW)], ewb)

      def row_body(j, carry2):
        pltpu.async_copy(table.at[srcb.at[j]], rows, gsem).wait()

        def mul_body(i, carry3):
          ridx = 2 * i + step
          v = plsc.load_gather(rows, [ridx, col])
          w = plsc.load_gather(ewb, [j * _IDXW + ridx])
          plsc.store_scatter(rows, [ridx, col], v * w)
          return carry3

        lax.fori_loop(0, n_vregs, mul_body, 0)

        pltpu.sync_copy(rows, acc_s.at[dstb.at[j]], add=True)
        if with_count:
          pltpu.sync_copy(onesv, cnt_s.at[dstb.at[j]], add=True)
        return carry2

      lax.fori_loop(0, _KROWS, row_body, 0)
      return carry

    lax.fori_loop(0, n_chunks, chunk_body, 0)

    plsc.subcore_barrier()

    copy_span(lambda o, r: acc_s.at[pl.ds(o, r)],
              lambda o, r: acc_o.at[c].at[pl.ds(o, r)])
    if with_count:
      @pl.when(c == 0)
      def _():
        copy_span(lambda o, r: cnt_s.at[pl.ds(o, r)],
                  lambda o, r: cnt0_o.at[pl.ds(o, r)])

      @pl.when(c == 1)
      def _():
        copy_span(lambda o, r: cnt_s.at[pl.ds(o, r)],
                  lambda o, r: cnt1_o.at[pl.ds(o, r)])

  return pl.kernel(
      body, out_type=out_type, mesh=mesh, scratch_types=scratch,
      compiler_params=pltpu.CompilerParams(
          needs_layout_passes=False, use_tc_tiling_on_sc=False))


def _tc_pre(x, w_l, w_r):
  def body(x_ref, wl_ref, wr_ref, y_ref, xr_ref):
    xv = x_ref[...]
    y_ref[...] = jnp.dot(xv, wl_ref[...], preferred_element_type=jnp.float32)
    xr_ref[...] = jnp.dot(xv, wr_ref[...], preferred_element_type=jnp.float32)

  n, din = x.shape
  dh = w_l.shape[1]
  blk = _TC_BLK
  grid = n // blk
  row = pl.BlockSpec((blk, din), lambda i: (i, 0))
  out = pl.BlockSpec((blk, dh), lambda i: (i, 0))
  full = pl.BlockSpec((din, dh), lambda i: (0, 0))
  return pl.pallas_call(
      body,
      grid=(grid,),
      in_specs=[row, full, full],
      out_specs=[out, out],
      out_shape=[jax.ShapeDtypeStruct((n, dh), jnp.float32)] * 2,
  )(x, w_l, w_r)


def _tc_mid(acc, cnt8a, cnt8b, xr, b, w3_l, w3_r):
  def body(acc_ref, ca_ref, cb_ref, xr_ref, b_ref, wl_ref, wr_ref,
           y2_ref, hr_ref, cnt_o_ref):
    ctot = ca_ref[...] + cb_ref[...]
    csafe = jnp.maximum(ctot, 1.0)
    h = (acc_ref[0] + acc_ref[1]) / csafe + b_ref[...] + xr_ref[...]
    h = jnp.maximum(h, 0.0)
    y2_ref[...] = jnp.dot(h, wl_ref[...], preferred_element_type=jnp.float32)
    hr_ref[...] = jnp.dot(h, wr_ref[...], preferred_element_type=jnp.float32)
    cnt_o_ref[...] = ctot

  n = acc.shape[1]
  dh = acc.shape[2]
  blk = _TC_BLK
  grid = n // blk
  rowp = pl.BlockSpec((_NCORES, blk, dh), lambda i: (0, i, 0))
  row = pl.BlockSpec((blk, dh), lambda i: (i, 0))
  bias = pl.BlockSpec((1, dh), lambda i: (0, 0))
  wfull = pl.BlockSpec((dh, dh), lambda i: (0, 0))
  return pl.pallas_call(
      body,
      grid=(grid,),
      in_specs=[rowp, row, row, row, bias, wfull, wfull],
      out_specs=[row, row, row],
      out_shape=[jax.ShapeDtypeStruct((n, dh), jnp.float32)] * 3,
  )(acc, cnt8a, cnt8b, xr, b, w3_l, w3_r)


def _tc_post(acc, cnt, hr, b):
  def body(acc_ref, cnt_ref, hr_ref, b_ref, out_ref):
    z = (acc_ref[0] + acc_ref[1]) / jnp.maximum(cnt_ref[...], 1.0)
    z = z + b_ref[...] + hr_ref[...]
    m = jnp.max(z, axis=1, keepdims=True)
    zs = z - m
    out_ref[...] = zs - jnp.log(jnp.sum(jnp.exp(zs), axis=1, keepdims=True))

  n = acc.shape[1]
  dh = acc.shape[2]
  blk = _TC_BLK
  grid = n // blk
  rowp = pl.BlockSpec((_NCORES, blk, dh), lambda i: (0, i, 0))
  row = pl.BlockSpec((blk, dh), lambda i: (i, 0))
  bias = pl.BlockSpec((1, dh), lambda i: (0, 0))
  return pl.pallas_call(
      body,
      grid=(grid,),
      in_specs=[rowp, row, row, bias],
      out_specs=row,
      out_shape=jax.ShapeDtypeStruct((n, dh), jnp.float32),
  )(acc, cnt, hr, b)


def kernel(x, edge_index, edge_weight, W1_l, b1, W1_r, W3_l, b3, W3_r):
  n, _ = x.shape
  dh = W1_l.shape[1]
  e = edge_weight.shape[0]

  rows_total = -(-e // _IDXW)
  rows_per_tile = -(-(-(-rows_total // _NTILES)) // _KROWS) * _KROWS
  ep = _NTILES * rows_per_tile * _IDXW
  pad = ep - e
  src = jnp.concatenate([edge_index[0], jnp.zeros((pad,), jnp.int32)])
  dst = jnp.concatenate([edge_index[1], jnp.full((pad,), n, jnp.int32)])
  ew = jnp.concatenate([edge_weight, jnp.zeros((pad,), jnp.float32)])
  src2d = src.reshape(-1, _IDXW)
  dst2d = dst.reshape(-1, _IDXW)
  z8 = jnp.zeros((n, dh), jnp.float32)
  z1 = jnp.zeros((n,), jnp.float32)

  sc_count = _make_sc_pass(n, dh, rows_per_tile, with_count=True)
  sc_plain = _make_sc_pass(n, dh, rows_per_tile, with_count=False)

  y1, xr = _tc_pre(x, W1_l, W1_r)
  acc1, cnt_a, cnt_b = sc_count(y1, src2d, dst2d, ew, z8, z1)
  cnt8a = jnp.broadcast_to(cnt_a[:, None], (n, dh))
  cnt8b = jnp.broadcast_to(cnt_b[:, None], (n, dh))
  y2, hr, cntc = _tc_mid(acc1, cnt8a, cnt8b, xr, b1.reshape(1, dh), W3_l, W3_r)

  (acc2,) = sc_plain(y2, src2d, dst2d, ew, z8)
  return _tc_post(acc2, cntc, hr, b3.reshape(1, dh))


# speedup vs baseline: 27.5706x; 1.9338x over previous
import functools

import jax
import jax.numpy as jnp
from jax import lax
from jax.experimental import pallas as pl
from jax.experimental.pallas import tpu as pltpu
from jax.experimental.pallas import tpu_sc as plsc

_LANES = 16
_IDXW = 128
_KROWS = 16
_NCORES = 2
_NSUB = 16
_NTILES = _NCORES * _NSUB
_TC_BLK = 5000


def _make_sc_pass(n_nodes, dh, rows_per_tile, with_count):
  assert rows_per_tile % _KROWS == 0
  n_chunks = rows_per_tile // _KROWS
  n_rows_total = _NTILES * rows_per_tile
  rows0 = (-(-n_nodes // _NSUB) + 7) // 8 * 8
  rows_last = n_nodes - (_NSUB - 1) * rows0
  assert rows_last > 0
  n_pad = n_nodes + 8

  mesh = plsc.VectorSubcoreMesh(core_axis_name="c", subcore_axis_name="s")

  out_type = [jax.ShapeDtypeStruct((_NCORES, n_nodes, dh), jnp.float32)]
  if with_count:
    out_type += [jax.ShapeDtypeStruct((n_nodes,), jnp.float32)] * 2

  ch = _KROWS * _IDXW
  scratch = [
      pltpu.VMEM((2, _KROWS, _IDXW), jnp.int32),
      pltpu.VMEM((2, _KROWS, _IDXW), jnp.int32),
      pltpu.VMEM((2, ch), jnp.float32),
      pltpu.VMEM((2, ch, dh), jnp.float32),
      pltpu.VMEM_SHARED((n_pad, dh), jnp.float32),
      pltpu.SemaphoreType.DMA,
      pltpu.SemaphoreType.DMA,
      pltpu.SemaphoreType.DMA,
      pltpu.SemaphoreType.DMA,
  ]
  if with_count:
    scratch += [
        pltpu.VMEM((_IDXW,), jnp.float32),
        pltpu.VMEM_SHARED((n_pad,), jnp.float32),
    ]

  def body(*refs):
    if with_count:
      (table, src_h, dst_h, ew_h, z8_h, z1_h, acc_o, cnt0_o, cnt1_o,
       srcb, dstb, ewb, rows, acc_s, gsem0, gsem1, ssem0, ssem1,
       onesv, cnt_s) = refs
    else:
      (table, src_h, dst_h, ew_h, z8_h, acc_o,
       srcb, dstb, ewb, rows, acc_s, gsem0, gsem1, ssem0, ssem1) = refs
      z1_h = onesv = cnt_s = None
    gsem = (gsem0, gsem1)
    ssem = (ssem0, ssem1)

    c = lax.axis_index("c")
    s = lax.axis_index("s")
    wid = c * _NSUB + s
    off0 = s * rows0

    def copy_span(get_src, get_dst):
      @pl.when(s < _NSUB - 1)
      def _():
        pltpu.sync_copy(get_src(off0, rows0), get_dst(off0, rows0))

      @pl.when(s == _NSUB - 1)
      def _():
        lo = (_NSUB - 1) * rows0
        pltpu.sync_copy(get_src(lo, rows_last), get_dst(lo, rows_last))

    copy_span(lambda o, r: z8_h.at[pl.ds(o, r)],
              lambda o, r: acc_s.at[pl.ds(o, r)])
    if with_count:
      copy_span(lambda o, r: z1_h.at[pl.ds(o, r)],
                lambda o, r: cnt_s.at[pl.ds(o, r)])
      for i in range(_IDXW // _LANES):
        onesv[pl.ds(i * _LANES, _LANES)] = jnp.full((_LANES,), 1.0, jnp.float32)

    plsc.subcore_barrier()

    iota = lax.iota(jnp.int32, _LANES)
    step = (iota >= (_LANES // 2)).astype(jnp.int32)
    col = jnp.bitwise_and(iota, dh - 1)
    n_vregs = _IDXW * dh // _LANES

    def load_idx(c, p):
      base = wid * rows_per_tile + c * _KROWS
      pltpu.sync_copy(src_h.at[pl.ds(base, _KROWS)], srcb.at[p])
      pltpu.sync_copy(dst_h.at[pl.ds(base, _KROWS)], dstb.at[p])
      pltpu.sync_copy(ew_h.at[pl.ds(base * _IDXW, ch)], ewb.at[p])

    def fire_gathers(p):
      def fg(j, carry):
        pltpu.async_copy(table.at[srcb.at[p].at[j]],
                         rows.at[p].at[pl.ds(j * _IDXW, _IDXW)], gsem[p])
        return carry
      lax.fori_loop(0, _KROWS, fg, 0)

    def drain_gathers(p):
      pltpu.make_async_copy(z8_h.at[pl.ds(0, ch)], rows.at[p], gsem[p]).wait()

    def fire_scatters(p):
      def fs(j, carry):
        pltpu.async_copy(rows.at[p].at[pl.ds(j * _IDXW, _IDXW)],
                         acc_s.at[dstb.at[p].at[j]], ssem[p], add=True)
        if with_count:
          pltpu.async_copy(onesv, cnt_s.at[dstb.at[p].at[j]], ssem[p],
                           add=True)
        return carry
      lax.fori_loop(0, _KROWS, fs, 0)

    def drain_scatters(p):
      pltpu.make_async_copy(z8_h.at[pl.ds(0, ch)], rows.at[p], ssem[p]).wait()
      if with_count:
        pltpu.make_async_copy(z1_h.at[pl.ds(0, ch)], ewb.at[p],
                              ssem[p]).wait()

    def multiply(p):
      rows_p = rows.at[p]
      ew_p = ewb.at[p]
      def mb(i, ridx):
        v = plsc.load_gather(rows_p, [ridx, col])
        w = plsc.load_gather(ew_p, [ridx])
        plsc.store_scatter(rows_p, [ridx, col], v * w)
        return ridx + 2
      lax.fori_loop(0, ch * dh // _LANES, mb, step)

    load_idx(0, 0)
    fire_gathers(0)
    load_idx(1, 1)
    fire_gathers(1)
    drain_gathers(0)
    multiply(0)
    fire_scatters(0)

    def steady(c, p):
      q = 1 - p
      drain_scatters(q)
      @pl.when(c + 1 < n_chunks)
      def _():
        load_idx(c + 1, q)
        fire_gathers(q)
      drain_gathers(p)
      multiply(p)
      fire_scatters(p)

    def pair_body(g, carry):
      steady(2 * g + 1, 1)
      steady(2 * g + 2, 0)
      return carry
    lax.fori_loop(0, (n_chunks - 1) // 2, pair_body, 0)
    if n_chunks % 2 == 0:
      steady(n_chunks - 1, 1)
    drain_scatters((n_chunks - 1) & 1)

    plsc.subcore_barrier()

    copy_span(lambda o, r: acc_s.at[pl.ds(o, r)],
              lambda o, r: acc_o.at[c].at[pl.ds(o, r)])
    if with_count:
      @pl.when(c == 0)
      def _():
        copy_span(lambda o, r: cnt_s.at[pl.ds(o, r)],
                  lambda o, r: cnt0_o.at[pl.ds(o, r)])

      @pl.when(c == 1)
      def _():
        copy_span(lambda o, r: cnt_s.at[pl.ds(o, r)],
                  lambda o, r: cnt1_o.at[pl.ds(o, r)])

  return pl.kernel(
      body, out_type=out_type, mesh=mesh, scratch_types=scratch,
      compiler_params=pltpu.CompilerParams(
          needs_layout_passes=False, use_tc_tiling_on_sc=False))


def _tc_pre(x, w_l, w_r):
  def body(x_ref, wl_ref, wr_ref, y_ref, xr_ref):
    xv = x_ref[...]
    y_ref[...] = jnp.dot(xv, wl_ref[...], preferred_element_type=jnp.float32)
    xr_ref[...] = jnp.dot(xv, wr_ref[...], preferred_element_type=jnp.float32)

  n, din = x.shape
  dh = w_l.shape[1]
  blk = _TC_BLK
  grid = n // blk
  row = pl.BlockSpec((blk, din), lambda i: (i, 0))
  out = pl.BlockSpec((blk, dh), lambda i: (i, 0))
  full = pl.BlockSpec((din, dh), lambda i: (0, 0))
  return pl.pallas_call(
      body,
      grid=(grid,),
      in_specs=[row, full, full],
      out_specs=[out, out],
      out_shape=[jax.ShapeDtypeStruct((n, dh), jnp.float32)] * 2,
  )(x, w_l, w_r)


def _tc_mid(acc, cnt8a, cnt8b, xr, b, w3_l, w3_r):
  def body(acc_ref, ca_ref, cb_ref, xr_ref, b_ref, wl_ref, wr_ref,
           y2_ref, hr_ref, cnt_o_ref):
    ctot = ca_ref[...] + cb_ref[...]
    csafe = jnp.maximum(ctot, 1.0)
    h = (acc_ref[0] + acc_ref[1]) / csafe + b_ref[...] + xr_ref[...]
    h = jnp.maximum(h, 0.0)
    y2_ref[...] = jnp.dot(h, wl_ref[...], preferred_element_type=jnp.float32)
    hr_ref[...] = jnp.dot(h, wr_ref[...], preferred_element_type=jnp.float32)
    cnt_o_ref[...] = ctot

  n = acc.shape[1]
  dh = acc.shape[2]
  blk = _TC_BLK
  grid = n // blk
  rowp = pl.BlockSpec((_NCORES, blk, dh), lambda i: (0, i, 0))
  row = pl.BlockSpec((blk, dh), lambda i: (i, 0))
  bias = pl.BlockSpec((1, dh), lambda i: (0, 0))
  wfull = pl.BlockSpec((dh, dh), lambda i: (0, 0))
  return pl.pallas_call(
      body,
      grid=(grid,),
      in_specs=[rowp, row, row, row, bias, wfull, wfull],
      out_specs=[row, row, row],
      out_shape=[jax.ShapeDtypeStruct((n, dh), jnp.float32)] * 3,
  )(acc, cnt8a, cnt8b, xr, b, w3_l, w3_r)


def _tc_post(acc, cnt, hr, b):
  def body(acc_ref, cnt_ref, hr_ref, b_ref, out_ref):
    z = (acc_ref[0] + acc_ref[1]) / jnp.maximum(cnt_ref[...], 1.0)
    z = z + b_ref[...] + hr_ref[...]
    m = jnp.max(z, axis=1, keepdims=True)
    zs = z - m
    out_ref[...] = zs - jnp.log(jnp.sum(jnp.exp(zs), axis=1, keepdims=True))

  n = acc.shape[1]
  dh = acc.shape[2]
  blk = _TC_BLK
  grid = n // blk
  rowp = pl.BlockSpec((_NCORES, blk, dh), lambda i: (0, i, 0))
  row = pl.BlockSpec((blk, dh), lambda i: (i, 0))
  bias = pl.BlockSpec((1, dh), lambda i: (0, 0))
  return pl.pallas_call(
      body,
      grid=(grid,),
      in_specs=[rowp, row, row, bias],
      out_specs=row,
      out_shape=jax.ShapeDtypeStruct((n, dh), jnp.float32),
  )(acc, cnt, hr, b)


def kernel(x, edge_index, edge_weight, W1_l, b1, W1_r, W3_l, b3, W3_r):
  n, _ = x.shape
  dh = W1_l.shape[1]
  e = edge_weight.shape[0]

  rows_total = -(-e // _IDXW)
  rows_per_tile = -(-(-(-rows_total // _NTILES)) // _KROWS) * _KROWS
  ep = _NTILES * rows_per_tile * _IDXW
  pad = ep - e
  src = jnp.concatenate([edge_index[0], jnp.zeros((pad,), jnp.int32)])
  dst = jnp.concatenate([edge_index[1], jnp.full((pad,), n, jnp.int32)])
  ew = jnp.concatenate([edge_weight, jnp.zeros((pad,), jnp.float32)])
  src2d = src.reshape(-1, _IDXW)
  dst2d = dst.reshape(-1, _IDXW)
  z8 = jnp.zeros((n, dh), jnp.float32)
  z1 = jnp.zeros((n,), jnp.float32)

  sc_count = _make_sc_pass(n, dh, rows_per_tile, with_count=True)
  sc_plain = _make_sc_pass(n, dh, rows_per_tile, with_count=False)

  y1, xr = _tc_pre(x, W1_l, W1_r)
  acc1, cnt_a, cnt_b = sc_count(y1, src2d, dst2d, ew, z8, z1)
  cnt8a = jnp.broadcast_to(cnt_a[:, None], (n, dh))
  cnt8b = jnp.broadcast_to(cnt_b[:, None], (n, dh))
  y2, hr, cntc = _tc_mid(acc1, cnt8a, cnt8b, xr, b1.reshape(1, dh), W3_l, W3_r)

  (acc2,) = sc_plain(y2, src2d, dst2d, ew, z8)
  return _tc_post(acc2, cntc, hr, b3.reshape(1, dh))


# speedup vs baseline: 30.0728x; 1.0908x over previous
import functools

import jax
import jax.numpy as jnp
from jax import lax
from jax.experimental import pallas as pl
from jax.experimental.pallas import tpu as pltpu
from jax.experimental.pallas import tpu_sc as plsc

_LANES = 16
_IDXW = 128
_KROWS = 16
_NCORES = 2
_NSUB = 16
_NTILES = _NCORES * _NSUB
_TC_BLK = 5000


def _make_sc_pass(n_nodes, dh, rows_per_tile, with_count):
  assert rows_per_tile % _KROWS == 0
  n_chunks = rows_per_tile // _KROWS
  n_rows_total = _NTILES * rows_per_tile
  rows0 = (-(-n_nodes // _NSUB) + 7) // 8 * 8
  rows_last = n_nodes - (_NSUB - 1) * rows0
  assert rows_last > 0
  n_pad = n_nodes + 8

  mesh = plsc.VectorSubcoreMesh(core_axis_name="c", subcore_axis_name="s")

  out_type = [jax.ShapeDtypeStruct((_NCORES, n_nodes, dh), jnp.float32)]
  if with_count:
    out_type += [jax.ShapeDtypeStruct((n_nodes,), jnp.float32)] * 2

  ch = _KROWS * _IDXW
  scratch = [
      pltpu.VMEM((2, _KROWS, _IDXW), jnp.int32),
      pltpu.VMEM((2, _KROWS, _IDXW), jnp.int32),
      pltpu.VMEM((2, ch), jnp.float32),
      pltpu.VMEM((2, ch, dh), jnp.float32),
      pltpu.VMEM_SHARED((n_pad, dh), jnp.float32),
      pltpu.SemaphoreType.DMA,
      pltpu.SemaphoreType.DMA,
      pltpu.SemaphoreType.DMA,
      pltpu.SemaphoreType.DMA,
      pltpu.SemaphoreType.DMA,
      pltpu.SemaphoreType.DMA,
  ]
  if with_count:
    scratch += [
        pltpu.VMEM((_IDXW,), jnp.float32),
        pltpu.VMEM_SHARED((n_pad,), jnp.float32),
    ]

  def body(*refs):
    if with_count:
      (table, src_h, dst_h, ew_h, z8_h, z1_h, acc_o, cnt0_o, cnt1_o,
       srcb, dstb, ewb, rows, acc_s, gsem0, gsem1, ssem0, ssem1,
       isem0, isem1, onesv, cnt_s) = refs
    else:
      (table, src_h, dst_h, ew_h, z8_h, acc_o,
       srcb, dstb, ewb, rows, acc_s, gsem0, gsem1, ssem0, ssem1,
       isem0, isem1) = refs
      z1_h = onesv = cnt_s = None
    gsem = (gsem0, gsem1)
    ssem = (ssem0, ssem1)
    isem = (isem0, isem1)

    c = lax.axis_index("c")
    s = lax.axis_index("s")
    wid = c * _NSUB + s
    off0 = s * rows0

    def copy_span(get_src, get_dst):
      @pl.when(s < _NSUB - 1)
      def _():
        pltpu.sync_copy(get_src(off0, rows0), get_dst(off0, rows0))

      @pl.when(s == _NSUB - 1)
      def _():
        lo = (_NSUB - 1) * rows0
        pltpu.sync_copy(get_src(lo, rows_last), get_dst(lo, rows_last))

    copy_span(lambda o, r: z8_h.at[pl.ds(o, r)],
              lambda o, r: acc_s.at[pl.ds(o, r)])
    if with_count:
      copy_span(lambda o, r: z1_h.at[pl.ds(o, r)],
                lambda o, r: cnt_s.at[pl.ds(o, r)])
      for i in range(_IDXW // _LANES):
        onesv[pl.ds(i * _LANES, _LANES)] = jnp.full((_LANES,), 1.0, jnp.float32)

    plsc.subcore_barrier()

    iota = lax.iota(jnp.int32, _LANES)
    step = (iota >= (_LANES // 2)).astype(jnp.int32)
    col = jnp.bitwise_and(iota, dh - 1)
    n_vregs = _IDXW * dh // _LANES

    def load_idx(c, p):
      base = wid * rows_per_tile + c * _KROWS
      pltpu.async_copy(src_h.at[pl.ds(base, _KROWS)], srcb.at[p], isem[p])
      pltpu.async_copy(dst_h.at[pl.ds(base, _KROWS)], dstb.at[p], isem[p])
      pltpu.async_copy(ew_h.at[pl.ds(base * _IDXW, ch)], ewb.at[p], isem[p])

    def drain_idx(p):
      pltpu.make_async_copy(src_h.at[pl.ds(0, _KROWS)], srcb.at[p],
                            isem[p]).wait()
      pltpu.make_async_copy(dst_h.at[pl.ds(0, _KROWS)], dstb.at[p],
                            isem[p]).wait()
      pltpu.make_async_copy(ew_h.at[pl.ds(0, ch)], ewb.at[p], isem[p]).wait()

    def fire_gathers(p):
      def fg(j, carry):
        pltpu.async_copy(table.at[srcb.at[p].at[j]],
                         rows.at[p].at[pl.ds(j * _IDXW, _IDXW)], gsem[p])
        return carry
      lax.fori_loop(0, _KROWS, fg, 0)

    def drain_gathers(p):
      pltpu.make_async_copy(z8_h.at[pl.ds(0, ch)], rows.at[p], gsem[p]).wait()

    def fire_scatters(p):
      def fs(j, carry):
        pltpu.async_copy(rows.at[p].at[pl.ds(j * _IDXW, _IDXW)],
                         acc_s.at[dstb.at[p].at[j]], ssem[p], add=True)
        if with_count:
          pltpu.async_copy(onesv, cnt_s.at[dstb.at[p].at[j]], ssem[p],
                           add=True)
        return carry
      lax.fori_loop(0, _KROWS, fs, 0)

    def drain_scatters(p):
      pltpu.make_async_copy(z8_h.at[pl.ds(0, ch)], rows.at[p], ssem[p]).wait()
      if with_count:
        pltpu.make_async_copy(z1_h.at[pl.ds(0, ch)], ewb.at[p],
                              ssem[p]).wait()

    def multiply(p):
      rows_p = rows.at[p]
      ew_p = ewb.at[p]
      def mb(i, ridx):
        v = plsc.load_gather(rows_p, [ridx, col])
        w = plsc.load_gather(ew_p, [ridx])
        plsc.store_scatter(rows_p, [ridx, col], v * w)
        return ridx + 2
      lax.fori_loop(0, ch * dh // _LANES, mb, step, unroll=8)

    load_idx(0, 0)
    load_idx(1, 1)
    drain_idx(0)
    fire_gathers(0)
    drain_idx(1)
    fire_gathers(1)
    drain_gathers(0)
    multiply(0)
    fire_scatters(0)

    def steady(c, p):
      q = 1 - p
      drain_scatters(q)
      @pl.when(c + 1 < n_chunks)
      def _():
        load_idx(c + 1, q)
      drain_gathers(p)
      @pl.when(c + 1 < n_chunks)
      def _():
        drain_idx(q)
        fire_gathers(q)
      multiply(p)
      fire_scatters(p)

    def pair_body(g, carry):
      steady(2 * g + 1, 1)
      steady(2 * g + 2, 0)
      return carry
    lax.fori_loop(0, (n_chunks - 1) // 2, pair_body, 0)
    if n_chunks % 2 == 0:
      steady(n_chunks - 1, 1)
    drain_scatters((n_chunks - 1) & 1)

    plsc.subcore_barrier()

    copy_span(lambda o, r: acc_s.at[pl.ds(o, r)],
              lambda o, r: acc_o.at[c].at[pl.ds(o, r)])
    if with_count:
      @pl.when(c == 0)
      def _():
        copy_span(lambda o, r: cnt_s.at[pl.ds(o, r)],
                  lambda o, r: cnt0_o.at[pl.ds(o, r)])

      @pl.when(c == 1)
      def _():
        copy_span(lambda o, r: cnt_s.at[pl.ds(o, r)],
                  lambda o, r: cnt1_o.at[pl.ds(o, r)])

  return pl.kernel(
      body, out_type=out_type, mesh=mesh, scratch_types=scratch,
      compiler_params=pltpu.CompilerParams(
          needs_layout_passes=False, use_tc_tiling_on_sc=False))


def _tc_pre(x, w_l, w_r):
  def body(x_ref, wl_ref, wr_ref, y_ref, xr_ref):
    xv = x_ref[...]
    y_ref[...] = jnp.dot(xv, wl_ref[...], preferred_element_type=jnp.float32)
    xr_ref[...] = jnp.dot(xv, wr_ref[...], preferred_element_type=jnp.float32)

  n, din = x.shape
  dh = w_l.shape[1]
  blk = _TC_BLK
  grid = n // blk
  row = pl.BlockSpec((blk, din), lambda i: (i, 0))
  out = pl.BlockSpec((blk, dh), lambda i: (i, 0))
  full = pl.BlockSpec((din, dh), lambda i: (0, 0))
  return pl.pallas_call(
      body,
      grid=(grid,),
      in_specs=[row, full, full],
      out_specs=[out, out],
      out_shape=[jax.ShapeDtypeStruct((n, dh), jnp.float32)] * 2,
  )(x, w_l, w_r)


def _tc_mid(acc, cnt8a, cnt8b, xr, b, w3_l, w3_r):
  def body(acc_ref, ca_ref, cb_ref, xr_ref, b_ref, wl_ref, wr_ref,
           y2_ref, hr_ref, cnt_o_ref):
    ctot = ca_ref[...] + cb_ref[...]
    csafe = jnp.maximum(ctot, 1.0)
    h = (acc_ref[0] + acc_ref[1]) / csafe + b_ref[...] + xr_ref[...]
    h = jnp.maximum(h, 0.0)
    y2_ref[...] = jnp.dot(h, wl_ref[...], preferred_element_type=jnp.float32)
    hr_ref[...] = jnp.dot(h, wr_ref[...], preferred_element_type=jnp.float32)
    cnt_o_ref[...] = ctot

  n = acc.shape[1]
  dh = acc.shape[2]
  blk = _TC_BLK
  grid = n // blk
  rowp = pl.BlockSpec((_NCORES, blk, dh), lambda i: (0, i, 0))
  row = pl.BlockSpec((blk, dh), lambda i: (i, 0))
  bias = pl.BlockSpec((1, dh), lambda i: (0, 0))
  wfull = pl.BlockSpec((dh, dh), lambda i: (0, 0))
  return pl.pallas_call(
      body,
      grid=(grid,),
      in_specs=[rowp, row, row, row, bias, wfull, wfull],
      out_specs=[row, row, row],
      out_shape=[jax.ShapeDtypeStruct((n, dh), jnp.float32)] * 3,
  )(acc, cnt8a, cnt8b, xr, b, w3_l, w3_r)


def _tc_post(acc, cnt, hr, b):
  def body(acc_ref, cnt_ref, hr_ref, b_ref, out_ref):
    z = (acc_ref[0] + acc_ref[1]) / jnp.maximum(cnt_ref[...], 1.0)
    z = z + b_ref[...] + hr_ref[...]
    m = jnp.max(z, axis=1, keepdims=True)
    zs = z - m
    out_ref[...] = zs - jnp.log(jnp.sum(jnp.exp(zs), axis=1, keepdims=True))

  n = acc.shape[1]
  dh = acc.shape[2]
  blk = _TC_BLK
  grid = n // blk
  rowp = pl.BlockSpec((_NCORES, blk, dh), lambda i: (0, i, 0))
  row = pl.BlockSpec((blk, dh), lambda i: (i, 0))
  bias = pl.BlockSpec((1, dh), lambda i: (0, 0))
  return pl.pallas_call(
      body,
      grid=(grid,),
      in_specs=[rowp, row, row, bias],
      out_specs=row,
      out_shape=jax.ShapeDtypeStruct((n, dh), jnp.float32),
  )(acc, cnt, hr, b)


def kernel(x, edge_index, edge_weight, W1_l, b1, W1_r, W3_l, b3, W3_r):
  n, _ = x.shape
  dh = W1_l.shape[1]
  e = edge_weight.shape[0]

  rows_total = -(-e // _IDXW)
  rows_per_tile = -(-(-(-rows_total // _NTILES)) // _KROWS) * _KROWS
  ep = _NTILES * rows_per_tile * _IDXW
  pad = ep - e
  src = jnp.concatenate([edge_index[0], jnp.zeros((pad,), jnp.int32)])
  dst = jnp.concatenate([edge_index[1], jnp.full((pad,), n, jnp.int32)])
  ew = jnp.concatenate([edge_weight, jnp.zeros((pad,), jnp.float32)])
  src2d = src.reshape(-1, _IDXW)
  dst2d = dst.reshape(-1, _IDXW)
  z8 = jnp.zeros((n, dh), jnp.float32)
  z1 = jnp.zeros((n,), jnp.float32)

  sc_count = _make_sc_pass(n, dh, rows_per_tile, with_count=True)
  sc_plain = _make_sc_pass(n, dh, rows_per_tile, with_count=False)

  y1, xr = _tc_pre(x, W1_l, W1_r)
  acc1, cnt_a, cnt_b = sc_count(y1, src2d, dst2d, ew, z8, z1)
  cnt8a = jnp.broadcast_to(cnt_a[:, None], (n, dh))
  cnt8b = jnp.broadcast_to(cnt_b[:, None], (n, dh))
  y2, hr, cntc = _tc_mid(acc1, cnt8a, cnt8b, xr, b1.reshape(1, dh), W3_l, W3_r)

  (acc2,) = sc_plain(y2, src2d, dst2d, ew, z8)
  return _tc_post(acc2, cntc, hr, b3.reshape(1, dh))


# speedup vs baseline: 43.2736x; 1.4390x over previous
import functools

import jax
import jax.numpy as jnp
from jax import lax
from jax.experimental import pallas as pl
from jax.experimental.pallas import tpu as pltpu
from jax.experimental.pallas import tpu_sc as plsc

_LANES = 16
_IDXW = 128
_KROWS = 16
_NCORES = 2
_NSUB = 16
_NTILES = _NCORES * _NSUB
_TC_BLK = 1024


def _make_sc_pass(n_nodes, dh, rows_per_tile, with_count):
  assert rows_per_tile % _KROWS == 0
  n_chunks = rows_per_tile // _KROWS
  n_rows_total = _NTILES * rows_per_tile
  rows0 = (-(-n_nodes // _NSUB) + 7) // 8 * 8
  rows_last = n_nodes - (_NSUB - 1) * rows0
  assert rows_last > 0
  n_pad = n_nodes + 8

  mesh = plsc.VectorSubcoreMesh(core_axis_name="c", subcore_axis_name="s")

  out_type = [jax.ShapeDtypeStruct((_NCORES, n_nodes, dh), jnp.float32)]
  if with_count:
    out_type += [jax.ShapeDtypeStruct((n_nodes,), jnp.float32)] * 2

  ch = _KROWS * _IDXW
  scratch = [
      pltpu.VMEM((2, _KROWS, _IDXW), jnp.int32),
      pltpu.VMEM((2, _KROWS, _IDXW), jnp.int32),
      pltpu.VMEM((2, ch), jnp.float32),
      pltpu.VMEM((2, ch, dh), jnp.float32),
      pltpu.VMEM_SHARED((n_pad, dh), jnp.float32),
      pltpu.SemaphoreType.DMA,
      pltpu.SemaphoreType.DMA,
      pltpu.SemaphoreType.DMA,
      pltpu.SemaphoreType.DMA,
      pltpu.SemaphoreType.DMA,
      pltpu.SemaphoreType.DMA,
  ]
  if with_count:
    scratch += [
        pltpu.VMEM((_IDXW,), jnp.float32),
        pltpu.VMEM_SHARED((n_pad,), jnp.float32),
    ]

  def body(*refs):
    if with_count:
      (table, src_h, dst_h, ew_h, z8_h, z1_h, acc_o, cnt0_o, cnt1_o,
       srcb, dstb, ewb, rows, acc_s, gsem0, gsem1, ssem0, ssem1,
       isem0, isem1, onesv, cnt_s) = refs
    else:
      (table, src_h, dst_h, ew_h, z8_h, acc_o,
       srcb, dstb, ewb, rows, acc_s, gsem0, gsem1, ssem0, ssem1,
       isem0, isem1) = refs
      z1_h = onesv = cnt_s = None
    gsem = (gsem0, gsem1)
    ssem = (ssem0, ssem1)
    isem = (isem0, isem1)

    c = lax.axis_index("c")
    s = lax.axis_index("s")
    wid = c * _NSUB + s
    off0 = s * rows0

    def copy_span(get_src, get_dst):
      @pl.when(s < _NSUB - 1)
      def _():
        pltpu.sync_copy(get_src(off0, rows0), get_dst(off0, rows0))

      @pl.when(s == _NSUB - 1)
      def _():
        lo = (_NSUB - 1) * rows0
        pltpu.sync_copy(get_src(lo, rows_last), get_dst(lo, rows_last))

    copy_span(lambda o, r: z8_h.at[pl.ds(o, r)],
              lambda o, r: acc_s.at[pl.ds(o, r)])
    if with_count:
      copy_span(lambda o, r: z1_h.at[pl.ds(o, r)],
                lambda o, r: cnt_s.at[pl.ds(o, r)])
      for i in range(_IDXW // _LANES):
        onesv[pl.ds(i * _LANES, _LANES)] = jnp.full((_LANES,), 1.0, jnp.float32)

    plsc.subcore_barrier()

    iota = lax.iota(jnp.int32, _LANES)
    step = (iota >= (_LANES // 2)).astype(jnp.int32)
    col = jnp.bitwise_and(iota, dh - 1)
    n_vregs = _IDXW * dh // _LANES

    def load_idx(c, p):
      base = wid * rows_per_tile + c * _KROWS
      pltpu.async_copy(src_h.at[pl.ds(base, _KROWS)], srcb.at[p], isem[p])
      pltpu.async_copy(dst_h.at[pl.ds(base, _KROWS)], dstb.at[p], isem[p])
      pltpu.async_copy(ew_h.at[pl.ds(base * _IDXW, ch)], ewb.at[p], isem[p])

    def drain_idx(p):
      pltpu.make_async_copy(src_h.at[pl.ds(0, _KROWS)], srcb.at[p],
                            isem[p]).wait()
      pltpu.make_async_copy(dst_h.at[pl.ds(0, _KROWS)], dstb.at[p],
                            isem[p]).wait()
      pltpu.make_async_copy(ew_h.at[pl.ds(0, ch)], ewb.at[p], isem[p]).wait()

    def fire_gathers(p):
      def fg(j, carry):
        pltpu.async_copy(table.at[srcb.at[p].at[j]],
                         rows.at[p].at[pl.ds(j * _IDXW, _IDXW)], gsem[p])
        return carry
      lax.fori_loop(0, _KROWS, fg, 0)

    def drain_gathers(p):
      pltpu.make_async_copy(z8_h.at[pl.ds(0, ch)], rows.at[p], gsem[p]).wait()

    def fire_scatters(p):
      def fs(j, carry):
        pltpu.async_copy(rows.at[p].at[pl.ds(j * _IDXW, _IDXW)],
                         acc_s.at[dstb.at[p].at[j]], ssem[p], add=True)
        if with_count:
          pltpu.async_copy(onesv, cnt_s.at[dstb.at[p].at[j]], ssem[p],
                           add=True)
        return carry
      lax.fori_loop(0, _KROWS, fs, 0)

    def drain_scatters(p):
      pltpu.make_async_copy(z8_h.at[pl.ds(0, ch)], rows.at[p], ssem[p]).wait()
      if with_count:
        pltpu.make_async_copy(z1_h.at[pl.ds(0, ch)], ewb.at[p],
                              ssem[p]).wait()

    def multiply(p):
      rows_p = rows.at[p]
      ew_p = ewb.at[p]
      def mb(i, ridx):
        v = plsc.load_gather(rows_p, [ridx, col])
        w = plsc.load_gather(ew_p, [ridx])
        plsc.store_scatter(rows_p, [ridx, col], v * w)
        return ridx + 2
      lax.fori_loop(0, ch * dh // _LANES, mb, step, unroll=8)

    load_idx(0, 0)
    load_idx(1, 1)
    drain_idx(0)
    fire_gathers(0)
    drain_idx(1)
    fire_gathers(1)
    drain_gathers(0)
    multiply(0)
    fire_scatters(0)

    def steady(c, p):
      q = 1 - p
      drain_scatters(q)
      @pl.when(c + 1 < n_chunks)
      def _():
        load_idx(c + 1, q)
      drain_gathers(p)
      @pl.when(c + 1 < n_chunks)
      def _():
        drain_idx(q)
        fire_gathers(q)
      multiply(p)
      fire_scatters(p)

    def pair_body(g, carry):
      steady(2 * g + 1, 1)
      steady(2 * g + 2, 0)
      return carry
    lax.fori_loop(0, (n_chunks - 1) // 2, pair_body, 0)
    if n_chunks % 2 == 0:
      steady(n_chunks - 1, 1)
    drain_scatters((n_chunks - 1) & 1)

    plsc.subcore_barrier()

    copy_span(lambda o, r: acc_s.at[pl.ds(o, r)],
              lambda o, r: acc_o.at[c].at[pl.ds(o, r)])
    if with_count:
      @pl.when(c == 0)
      def _():
        copy_span(lambda o, r: cnt_s.at[pl.ds(o, r)],
                  lambda o, r: cnt0_o.at[pl.ds(o, r)])

      @pl.when(c == 1)
      def _():
        copy_span(lambda o, r: cnt_s.at[pl.ds(o, r)],
                  lambda o, r: cnt1_o.at[pl.ds(o, r)])

  return pl.kernel(
      body, out_type=out_type, mesh=mesh, scratch_types=scratch,
      compiler_params=pltpu.CompilerParams(
          needs_layout_passes=False, use_tc_tiling_on_sc=False))


def _tc_pre(x256, wbd_l, wbd_r):
  def body(x_ref, wl_ref, wr_ref, y_ref, xr_ref):
    xv = x_ref[...]
    y_ref[...] = jnp.dot(xv, wl_ref[...], preferred_element_type=jnp.float32)
    xr_ref[...] = jnp.dot(xv, wr_ref[...], preferred_element_type=jnp.float32)

  r, k = x256.shape
  blk = _TC_BLK
  grid = -(-r // blk)
  row_in = pl.BlockSpec((blk, k), lambda i: (i, 0))
  row = pl.BlockSpec((blk, 128), lambda i: (i, 0))
  wfull = pl.BlockSpec((k, 128), lambda i: (0, 0))
  return pl.pallas_call(
      body,
      grid=(grid,),
      in_specs=[row_in, wfull, wfull],
      out_specs=[row, row],
      out_shape=[jax.ShapeDtypeStruct((r, 128), jnp.float32)] * 2,
  )(x256, wbd_l, wbd_r)


def _tc_mid(accg, ca, cb, xrg, b128, wbd_l, wbd_r):
  def body(acc_ref, ca_ref, cb_ref, xr_ref, b_ref, wl_ref, wr_ref,
           y2_ref, hr_ref, ct_ref):
    ctot = ca_ref[...] + cb_ref[...]
    h = (acc_ref[0] + acc_ref[1]) / jnp.maximum(ctot, 1.0)
    h = jnp.maximum(h + b_ref[...] + xr_ref[...], 0.0)
    y2_ref[...] = jnp.dot(h, wl_ref[...], preferred_element_type=jnp.float32)
    hr_ref[...] = jnp.dot(h, wr_ref[...], preferred_element_type=jnp.float32)
    ct_ref[...] = ctot

  r = accg.shape[1]
  blk = _TC_BLK
  grid = -(-r // blk)
  rowp = pl.BlockSpec((_NCORES, blk, 128), lambda i: (0, i, 0))
  row = pl.BlockSpec((blk, 128), lambda i: (i, 0))
  bias = pl.BlockSpec((1, 128), lambda i: (0, 0))
  wfull = pl.BlockSpec((128, 128), lambda i: (0, 0))
  return pl.pallas_call(
      body,
      grid=(grid,),
      in_specs=[rowp, row, row, row, bias, wfull, wfull],
      out_specs=[row, row, row],
      out_shape=[jax.ShapeDtypeStruct((r, 128), jnp.float32)] * 3,
  )(accg, ca, cb, xrg, b128, wbd_l, wbd_r)


def _tc_post(accg, ctg, hrg, b128, ksum, p4, p2, p1):
  def body(acc_ref, ct_ref, hr_ref, b_ref, ks_ref, p4_ref, p2_ref, p1_ref,
           out_ref):
    z = (acc_ref[0] + acc_ref[1]) / jnp.maximum(ct_ref[...], 1.0)
    z = z + b_ref[...] + hr_ref[...]
    m = jnp.maximum(z, jnp.dot(z, p4_ref[...],
                               preferred_element_type=jnp.float32))
    m = jnp.maximum(m, jnp.dot(m, p2_ref[...],
                               preferred_element_type=jnp.float32))
    m = jnp.maximum(m, jnp.dot(m, p1_ref[...],
                               preferred_element_type=jnp.float32))
    e = jnp.exp(z - m)
    gs = jnp.dot(e, ks_ref[...], preferred_element_type=jnp.float32)
    out_ref[...] = z - m - jnp.log(gs)

  r = accg.shape[1]
  blk = _TC_BLK
  grid = -(-r // blk)
  rowp = pl.BlockSpec((_NCORES, blk, 128), lambda i: (0, i, 0))
  row = pl.BlockSpec((blk, 128), lambda i: (i, 0))
  bias = pl.BlockSpec((1, 128), lambda i: (0, 0))
  wfull = pl.BlockSpec((128, 128), lambda i: (0, 0))
  return pl.pallas_call(
      body,
      grid=(grid,),
      in_specs=[rowp, row, row, bias, wfull, wfull, wfull, wfull],
      out_specs=row,
      out_shape=jax.ShapeDtypeStruct((r, 128), jnp.float32),
  )(accg, ctg, hrg, b128, ksum, p4, p2, p1)


def kernel(x, edge_index, edge_weight, W1_l, b1, W1_r, W3_l, b3, W3_r):
  n, din = x.shape
  dh = W1_l.shape[1]
  e = edge_weight.shape[0]
  pack = 128 // dh
  r8 = n * dh // 128

  rows_total = -(-e // _IDXW)
  rows_per_tile = -(-(-(-rows_total // _NTILES)) // _KROWS) * _KROWS
  ep = _NTILES * rows_per_tile * _IDXW
  pad = ep - e
  src = jnp.concatenate([edge_index[0], jnp.zeros((pad,), jnp.int32)])
  dst = jnp.concatenate([edge_index[1], jnp.full((pad,), n, jnp.int32)])
  ew = jnp.concatenate([edge_weight, jnp.zeros((pad,), jnp.float32)])
  src2d = src.reshape(-1, _IDXW)
  dst2d = dst.reshape(-1, _IDXW)
  z8 = jnp.zeros((n, dh), jnp.float32)
  z1 = jnp.zeros((n,), jnp.float32)

  eye = jnp.eye(pack, dtype=jnp.float32)
  w1bd_l = jnp.kron(eye, W1_l)
  w1bd_r = jnp.kron(eye, W1_r)
  w3bd_l = jnp.kron(eye, W3_l)
  w3bd_r = jnp.kron(eye, W3_r)
  ksum = jnp.kron(eye, jnp.ones((dh, dh), jnp.float32))
  lane = jnp.arange(128)
  g, p = lane // dh, lane % dh
  def perm(k):
    m = jnp.zeros((128, 128), jnp.float32)
    return m.at[g * dh + (p + k) % dh, lane].set(1.0)
  p4, p2, p1 = perm(4), perm(2), perm(1)
  b1_128 = jnp.tile(b1, pack).reshape(1, 128)
  b3_128 = jnp.tile(b3, pack).reshape(1, 128)

  sc_count = _make_sc_pass(n, dh, rows_per_tile, with_count=True)
  sc_plain = _make_sc_pass(n, dh, rows_per_tile, with_count=False)

  y1g, xrg = _tc_pre(x.reshape(n // pack, pack * din), w1bd_l, w1bd_r)
  acc1, cnt_a, cnt_b = sc_count(y1g.reshape(n, dh), src2d, dst2d, ew, z8, z1)
  ca = jnp.broadcast_to(cnt_a[:, None], (n, dh)).reshape(r8, 128)
  cb = jnp.broadcast_to(cnt_b[:, None], (n, dh)).reshape(r8, 128)
  y2g, hrg, ctg = _tc_mid(acc1.reshape(_NCORES, r8, 128), ca, cb, xrg,
                          b1_128, w3bd_l, w3bd_r)

  (acc2,) = sc_plain(y2g.reshape(n, dh), src2d, dst2d, ew, z8)
  out = _tc_post(acc2.reshape(_NCORES, r8, 128), ctg, hrg, b3_128,
                 ksum, p4, p2, p1)
  return out.reshape(n, dh)


# speedup vs baseline: 44.8294x; 1.0360x over previous
import functools

import jax
import jax.numpy as jnp
from jax import lax
from jax.experimental import pallas as pl
from jax.experimental.pallas import tpu as pltpu
from jax.experimental.pallas import tpu_sc as plsc

_LANES = 16
_IDXW = 128
_KROWS = 16
_NCORES = 2
_NSUB = 16
_NTILES = _NCORES * _NSUB
_TC_BLK = 1024


def _make_sc_pass(n_nodes, dh, rows_total, with_count):
  base_rows = rows_total // _NTILES
  rem_rows = rows_total % _NTILES
  kr = max(d for d in range(16, 0, -1) if base_rows % d == 0)
  n_chunks = base_rows // kr
  rows0 = (-(-n_nodes // _NSUB) + 7) // 8 * 8
  rows_last = n_nodes - (_NSUB - 1) * rows0
  assert rows_last > 0
  n_pad = n_nodes + 8

  mesh = plsc.VectorSubcoreMesh(core_axis_name="c", subcore_axis_name="s")

  out_type = [jax.ShapeDtypeStruct((_NCORES, n_nodes, dh), jnp.float32)]
  if with_count:
    out_type += [jax.ShapeDtypeStruct((n_nodes,), jnp.float32)] * 2

  ch = kr * _IDXW
  scratch = [
      pltpu.VMEM((2, kr, _IDXW), jnp.int32),
      pltpu.VMEM((2, kr, _IDXW), jnp.int32),
      pltpu.VMEM((2, ch), jnp.float32),
      pltpu.VMEM((2, ch, dh), jnp.float32),
      pltpu.VMEM_SHARED((n_pad, dh), jnp.float32),
      pltpu.SemaphoreType.DMA,
      pltpu.SemaphoreType.DMA,
      pltpu.SemaphoreType.DMA,
      pltpu.SemaphoreType.DMA,
      pltpu.SemaphoreType.DMA,
      pltpu.SemaphoreType.DMA,
  ]
  if with_count:
    scratch += [
        pltpu.VMEM((_IDXW,), jnp.float32),
        pltpu.VMEM_SHARED((n_pad,), jnp.float32),
    ]

  def body(*refs):
    if with_count:
      (table, edge_h, ew_h, z8_h, z1_h, acc_o, cnt0_o, cnt1_o,
       srcb, dstb, ewb, rows, acc_s, gsem0, gsem1, ssem0, ssem1,
       isem0, isem1, onesv, cnt_s) = refs
    else:
      (table, edge_h, ew_h, z8_h, acc_o,
       srcb, dstb, ewb, rows, acc_s, gsem0, gsem1, ssem0, ssem1,
       isem0, isem1) = refs
      z1_h = onesv = cnt_s = None
    src_h = edge_h.at[0]
    dst_h = edge_h.at[1]
    gsem = (gsem0, gsem1)
    ssem = (ssem0, ssem1)
    isem = (isem0, isem1)

    c = lax.axis_index("c")
    s = lax.axis_index("s")
    wid = c * _NSUB + s
    start_row = wid * base_rows + jnp.minimum(wid, rem_rows)
    off0 = s * rows0

    def copy_span(get_src, get_dst):
      @pl.when(s < _NSUB - 1)
      def _():
        pltpu.sync_copy(get_src(off0, rows0), get_dst(off0, rows0))

      @pl.when(s == _NSUB - 1)
      def _():
        lo = (_NSUB - 1) * rows0
        pltpu.sync_copy(get_src(lo, rows_last), get_dst(lo, rows_last))

    copy_span(lambda o, r: z8_h.at[pl.ds(o, r)],
              lambda o, r: acc_s.at[pl.ds(o, r)])
    if with_count:
      copy_span(lambda o, r: z1_h.at[pl.ds(o, r)],
                lambda o, r: cnt_s.at[pl.ds(o, r)])
      for i in range(_IDXW // _LANES):
        onesv[pl.ds(i * _LANES, _LANES)] = jnp.full((_LANES,), 1.0, jnp.float32)

    plsc.subcore_barrier()

    iota = lax.iota(jnp.int32, _LANES)
    step = (iota >= (_LANES // 2)).astype(jnp.int32)
    col = jnp.bitwise_and(iota, dh - 1)
    n_vregs = _IDXW * dh // _LANES

    def load_idx(c, p):
      base = start_row + c * kr
      pltpu.async_copy(src_h.at[pl.ds(base, kr)], srcb.at[p], isem[p])
      pltpu.async_copy(dst_h.at[pl.ds(base, kr)], dstb.at[p], isem[p])
      pltpu.async_copy(ew_h.at[pl.ds(base * _IDXW, ch)], ewb.at[p], isem[p])

    def drain_idx(p):
      pltpu.make_async_copy(src_h.at[pl.ds(0, kr)], srcb.at[p],
                            isem[p]).wait()
      pltpu.make_async_copy(dst_h.at[pl.ds(0, kr)], dstb.at[p],
                            isem[p]).wait()
      pltpu.make_async_copy(ew_h.at[pl.ds(0, ch)], ewb.at[p], isem[p]).wait()

    def fire_gathers(p):
      def fg(j, carry):
        pltpu.async_copy(table.at[srcb.at[p].at[j]],
                         rows.at[p].at[pl.ds(j * _IDXW, _IDXW)], gsem[p])
        return carry
      lax.fori_loop(0, kr, fg, 0)

    def drain_gathers(p):
      pltpu.make_async_copy(z8_h.at[pl.ds(0, ch)], rows.at[p], gsem[p]).wait()

    def fire_scatters(p):
      def fs(j, carry):
        pltpu.async_copy(rows.at[p].at[pl.ds(j * _IDXW, _IDXW)],
                         acc_s.at[dstb.at[p].at[j]], ssem[p], add=True)
        if with_count:
          pltpu.async_copy(onesv, cnt_s.at[dstb.at[p].at[j]], ssem[p],
                           add=True)
        return carry
      lax.fori_loop(0, kr, fs, 0)

    def drain_scatters(p):
      pltpu.make_async_copy(z8_h.at[pl.ds(0, ch)], rows.at[p], ssem[p]).wait()
      if with_count:
        pltpu.make_async_copy(z1_h.at[pl.ds(0, ch)], ewb.at[p],
                              ssem[p]).wait()

    def multiply(p):
      rows_p = rows.at[p]
      ew_p = ewb.at[p]
      def mb(i, ridx):
        v = plsc.load_gather(rows_p, [ridx, col])
        w = plsc.load_gather(ew_p, [ridx])
        plsc.store_scatter(rows_p, [ridx, col], v * w)
        return ridx + 2
      lax.fori_loop(0, ch * dh // _LANES, mb, step, unroll=8)

    load_idx(0, 0)
    load_idx(1, 1)
    drain_idx(0)
    fire_gathers(0)
    drain_idx(1)
    fire_gathers(1)
    drain_gathers(0)
    multiply(0)
    fire_scatters(0)

    def steady(c, p):
      q = 1 - p
      drain_scatters(q)
      @pl.when(c + 1 < n_chunks)
      def _():
        load_idx(c + 1, q)
      drain_gathers(p)
      @pl.when(c + 1 < n_chunks)
      def _():
        drain_idx(q)
        fire_gathers(q)
      multiply(p)
      fire_scatters(p)

    def pair_body(g, carry):
      steady(2 * g + 1, 1)
      steady(2 * g + 2, 0)
      return carry
    lax.fori_loop(0, (n_chunks - 1) // 2, pair_body, 0)
    if n_chunks % 2 == 0:
      steady(n_chunks - 1, 1)
    drain_scatters((n_chunks - 1) & 1)

    if rem_rows:
      @pl.when(wid < rem_rows)
      def _():
        row = start_row + base_rows
        pltpu.sync_copy(src_h.at[pl.ds(row, 1)], srcb.at[0].at[pl.ds(0, 1)])
        pltpu.sync_copy(dst_h.at[pl.ds(row, 1)], dstb.at[0].at[pl.ds(0, 1)])
        pltpu.sync_copy(ew_h.at[pl.ds(row * _IDXW, _IDXW)],
                        ewb.at[0].at[pl.ds(0, _IDXW)])
        pltpu.async_copy(table.at[srcb.at[0].at[0]],
                         rows.at[0].at[pl.ds(0, _IDXW)], gsem[0]).wait()
        rows_t = rows.at[0]
        ew_t = ewb.at[0]
        def mb_t(i, ridx):
          v = plsc.load_gather(rows_t, [ridx, col])
          w = plsc.load_gather(ew_t, [ridx])
          plsc.store_scatter(rows_t, [ridx, col], v * w)
          return ridx + 2
        lax.fori_loop(0, _IDXW * dh // _LANES, mb_t, step)
        pltpu.sync_copy(rows.at[0].at[pl.ds(0, _IDXW)],
                        acc_s.at[dstb.at[0].at[0]], add=True)
        if with_count:
          pltpu.sync_copy(onesv, cnt_s.at[dstb.at[0].at[0]], add=True)

    plsc.subcore_barrier()

    copy_span(lambda o, r: acc_s.at[pl.ds(o, r)],
              lambda o, r: acc_o.at[c].at[pl.ds(o, r)])
    if with_count:
      @pl.when(c == 0)
      def _():
        copy_span(lambda o, r: cnt_s.at[pl.ds(o, r)],
                  lambda o, r: cnt0_o.at[pl.ds(o, r)])

      @pl.when(c == 1)
      def _():
        copy_span(lambda o, r: cnt_s.at[pl.ds(o, r)],
                  lambda o, r: cnt1_o.at[pl.ds(o, r)])

  return pl.kernel(
      body, out_type=out_type, mesh=mesh, scratch_types=scratch,
      compiler_params=pltpu.CompilerParams(
          needs_layout_passes=False, use_tc_tiling_on_sc=False))


def _tc_pre(x256, wbd_l, wbd_r):
  def body(x_ref, wl_ref, wr_ref, y_ref, xr_ref):
    xv = x_ref[...]
    y_ref[...] = jnp.dot(xv, wl_ref[...], preferred_element_type=jnp.float32)
    xr_ref[...] = jnp.dot(xv, wr_ref[...], preferred_element_type=jnp.float32)

  r, k = x256.shape
  blk = _TC_BLK
  grid = -(-r // blk)
  row_in = pl.BlockSpec((blk, k), lambda i: (i, 0))
  row = pl.BlockSpec((blk, 128), lambda i: (i, 0))
  wfull = pl.BlockSpec((k, 128), lambda i: (0, 0))
  return pl.pallas_call(
      body,
      grid=(grid,),
      in_specs=[row_in, wfull, wfull],
      out_specs=[row, row],
      out_shape=[jax.ShapeDtypeStruct((r, 128), jnp.float32)] * 2,
  )(x256, wbd_l, wbd_r)


def _tc_mid(accg, ca, cb, xrg, b128, wbd_l, wbd_r):
  def body(acc_ref, ca_ref, cb_ref, xr_ref, b_ref, wl_ref, wr_ref,
           y2_ref, hr_ref, ct_ref):
    ctot = ca_ref[...] + cb_ref[...]
    h = (acc_ref[0] + acc_ref[1]) / jnp.maximum(ctot, 1.0)
    h = jnp.maximum(h + b_ref[...] + xr_ref[...], 0.0)
    y2_ref[...] = jnp.dot(h, wl_ref[...], preferred_element_type=jnp.float32)
    hr_ref[...] = jnp.dot(h, wr_ref[...], preferred_element_type=jnp.float32)
    ct_ref[...] = ctot

  r = accg.shape[1]
  blk = _TC_BLK
  grid = -(-r // blk)
  rowp = pl.BlockSpec((_NCORES, blk, 128), lambda i: (0, i, 0))
  row = pl.BlockSpec((blk, 128), lambda i: (i, 0))
  bias = pl.BlockSpec((1, 128), lambda i: (0, 0))
  wfull = pl.BlockSpec((128, 128), lambda i: (0, 0))
  return pl.pallas_call(
      body,
      grid=(grid,),
      in_specs=[rowp, row, row, row, bias, wfull, wfull],
      out_specs=[row, row, row],
      out_shape=[jax.ShapeDtypeStruct((r, 128), jnp.float32)] * 3,
  )(accg, ca, cb, xrg, b128, wbd_l, wbd_r)


def _tc_post(accg, ctg, hrg, b128, ksum, p4, p2, p1):
  def body(acc_ref, ct_ref, hr_ref, b_ref, ks_ref, p4_ref, p2_ref, p1_ref,
           out_ref):
    z = (acc_ref[0] + acc_ref[1]) / jnp.maximum(ct_ref[...], 1.0)
    z = z + b_ref[...] + hr_ref[...]
    m = jnp.maximum(z, jnp.dot(z, p4_ref[...],
                               preferred_element_type=jnp.float32))
    m = jnp.maximum(m, jnp.dot(m, p2_ref[...],
                               preferred_element_type=jnp.float32))
    m = jnp.maximum(m, jnp.dot(m, p1_ref[...],
                               preferred_element_type=jnp.float32))
    e = jnp.exp(z - m)
    gs = jnp.dot(e, ks_ref[...], preferred_element_type=jnp.float32)
    out_ref[...] = z - m - jnp.log(gs)

  r = accg.shape[1]
  blk = _TC_BLK
  grid = -(-r // blk)
  rowp = pl.BlockSpec((_NCORES, blk, 128), lambda i: (0, i, 0))
  row = pl.BlockSpec((blk, 128), lambda i: (i, 0))
  bias = pl.BlockSpec((1, 128), lambda i: (0, 0))
  wfull = pl.BlockSpec((128, 128), lambda i: (0, 0))
  return pl.pallas_call(
      body,
      grid=(grid,),
      in_specs=[rowp, row, row, bias, wfull, wfull, wfull, wfull],
      out_specs=row,
      out_shape=jax.ShapeDtypeStruct((r, 128), jnp.float32),
  )(accg, ctg, hrg, b128, ksum, p4, p2, p1)


def kernel(x, edge_index, edge_weight, W1_l, b1, W1_r, W3_l, b3, W3_r):
  n, din = x.shape
  dh = W1_l.shape[1]
  e = edge_weight.shape[0]
  pack = 128 // dh
  r8 = n * dh // 128

  rows_total = -(-e // _IDXW)
  pad = rows_total * _IDXW - e
  if pad:
    edge_index = jnp.concatenate(
        [edge_index,
         jnp.stack([jnp.zeros((pad,), jnp.int32),
                    jnp.full((pad,), n, jnp.int32)])], axis=1)
    edge_weight = jnp.concatenate(
        [edge_weight, jnp.zeros((pad,), jnp.float32)])
  edge3d = edge_index.reshape(2, rows_total, _IDXW)
  ew = edge_weight
  z8 = jnp.zeros((n, dh), jnp.float32)
  z1 = jnp.zeros((n,), jnp.float32)

  eye = jnp.eye(pack, dtype=jnp.float32)
  w1bd_l = jnp.kron(eye, W1_l)
  w1bd_r = jnp.kron(eye, W1_r)
  w3bd_l = jnp.kron(eye, W3_l)
  w3bd_r = jnp.kron(eye, W3_r)
  ksum = jnp.kron(eye, jnp.ones((dh, dh), jnp.float32))
  lane = jnp.arange(128)
  g, p = lane // dh, lane % dh
  def perm(k):
    m = jnp.zeros((128, 128), jnp.float32)
    return m.at[g * dh + (p + k) % dh, lane].set(1.0)
  p4, p2, p1 = perm(4), perm(2), perm(1)
  b1_128 = jnp.tile(b1, pack).reshape(1, 128)
  b3_128 = jnp.tile(b3, pack).reshape(1, 128)

  sc_count = _make_sc_pass(n, dh, rows_total, with_count=True)
  sc_plain = _make_sc_pass(n, dh, rows_total, with_count=False)

  y1g, xrg = _tc_pre(x.reshape(n // pack, pack * din), w1bd_l, w1bd_r)
  acc1, cnt_a, cnt_b = sc_count(y1g.reshape(n, dh), edge3d, ew, z8, z1)
  ca = jnp.broadcast_to(cnt_a[:, None], (n, dh)).reshape(r8, 128)
  cb = jnp.broadcast_to(cnt_b[:, None], (n, dh)).reshape(r8, 128)
  y2g, hrg, ctg = _tc_mid(acc1.reshape(_NCORES, r8, 128), ca, cb, xrg,
                          b1_128, w3bd_l, w3bd_r)

  (acc2,) = sc_plain(y2g.reshape(n, dh), edge3d, ew, z8)
  out = _tc_post(acc2.reshape(_NCORES, r8, 128), ctg, hrg, b3_128,
                 ksum, p4, p2, p1)
  return out.reshape(n, dh)


# speedup vs baseline: 44.9004x; 1.0016x over previous
import jax
import jax.numpy as jnp
from jax import lax
from jax.experimental import pallas as pl
from jax.experimental.pallas import tpu as pltpu
from jax.experimental.pallas import tpu_sc as plsc

_LANES = 16
_IDXW = 128
_NCORES = 2
_NSUB = 16
_NTILES = _NCORES * _NSUB
_TC_BLK = 1024


def _make_sc_pass(n_nodes, dh, rows_total, with_count):
  base_rows = rows_total // _NTILES
  rem_rows = rows_total % _NTILES
  kr = max(d for d in range(16, 0, -1) if base_rows % d == 0)
  n_chunks = base_rows // kr
  rows0 = (-(-n_nodes // _NSUB) + 7) // 8 * 8
  rows_last = n_nodes - (_NSUB - 1) * rows0
  assert rows_last > 0
  n_pad = n_nodes + 8

  mesh = plsc.VectorSubcoreMesh(core_axis_name="c", subcore_axis_name="s")

  out_type = [jax.ShapeDtypeStruct((_NCORES, n_nodes, dh), jnp.float32)]
  if with_count:
    out_type += [jax.ShapeDtypeStruct((n_nodes,), jnp.float32)] * 2

  ch = kr * _IDXW
  scratch = [
      pltpu.VMEM((2, kr, _IDXW), jnp.int32),
      pltpu.VMEM((2, kr, _IDXW), jnp.int32),
      pltpu.VMEM((2, ch), jnp.float32),
      pltpu.VMEM((2, ch, dh), jnp.float32),
      pltpu.VMEM_SHARED((n_pad, dh), jnp.float32),
      pltpu.SemaphoreType.DMA,
      pltpu.SemaphoreType.DMA,
      pltpu.SemaphoreType.DMA,
      pltpu.SemaphoreType.DMA,
      pltpu.SemaphoreType.DMA,
      pltpu.SemaphoreType.DMA,
  ]
  if with_count:
    scratch += [
        pltpu.VMEM((_IDXW,), jnp.float32),
        pltpu.VMEM_SHARED((n_pad,), jnp.float32),
    ]

  def body(*refs):
    if with_count:
      (table, edge_h, ew_h, z8_h, z1_h, acc_o, cnt0_o, cnt1_o,
       srcb, dstb, ewb, rows, acc_s, gsem0, gsem1, ssem0, ssem1,
       isem0, isem1, onesv, cnt_s) = refs
    else:
      (table, edge_h, ew_h, z8_h, acc_o,
       srcb, dstb, ewb, rows, acc_s, gsem0, gsem1, ssem0, ssem1,
       isem0, isem1) = refs
      z1_h = onesv = cnt_s = None
    src_h = edge_h.at[0]
    dst_h = edge_h.at[1]
    gsem = (gsem0, gsem1)
    ssem = (ssem0, ssem1)
    isem = (isem0, isem1)

    c = lax.axis_index("c")
    s = lax.axis_index("s")
    wid = c * _NSUB + s
    start_row = wid * base_rows + jnp.minimum(wid, rem_rows)
    off0 = s * rows0

    def copy_span(get_src, get_dst):
      @pl.when(s < _NSUB - 1)
      def _():
        pltpu.sync_copy(get_src(off0, rows0), get_dst(off0, rows0))

      @pl.when(s == _NSUB - 1)
      def _():
        lo = (_NSUB - 1) * rows0
        pltpu.sync_copy(get_src(lo, rows_last), get_dst(lo, rows_last))

    copy_span(lambda o, r: z8_h.at[pl.ds(o, r)],
              lambda o, r: acc_s.at[pl.ds(o, r)])
    if with_count:
      copy_span(lambda o, r: z1_h.at[pl.ds(o, r)],
                lambda o, r: cnt_s.at[pl.ds(o, r)])
      for i in range(_IDXW // _LANES):
        onesv[pl.ds(i * _LANES, _LANES)] = jnp.full((_LANES,), 1.0, jnp.float32)

    plsc.subcore_barrier()

    iota = lax.iota(jnp.int32, _LANES)
    step = (iota >= (_LANES // 2)).astype(jnp.int32)
    col = jnp.bitwise_and(iota, dh - 1)
    n_vregs = _IDXW * dh // _LANES

    def load_idx(c, p):
      base = start_row + c * kr
      pltpu.async_copy(src_h.at[pl.ds(base, kr)], srcb.at[p], isem[p])
      pltpu.async_copy(dst_h.at[pl.ds(base, kr)], dstb.at[p], isem[p])
      pltpu.async_copy(ew_h.at[pl.ds(base * _IDXW, ch)], ewb.at[p], isem[p])

    def drain_idx(p):
      pltpu.make_async_copy(src_h.at[pl.ds(0, kr)], srcb.at[p],
                            isem[p]).wait()
      pltpu.make_async_copy(dst_h.at[pl.ds(0, kr)], dstb.at[p],
                            isem[p]).wait()
      pltpu.make_async_copy(ew_h.at[pl.ds(0, ch)], ewb.at[p], isem[p]).wait()

    def fire_gathers(p):
      def fg(j, carry):
        pltpu.async_copy(table.at[srcb.at[p].at[j]],
                         rows.at[p].at[pl.ds(j * _IDXW, _IDXW)], gsem[p])
        return carry
      lax.fori_loop(0, kr, fg, 0)

    def drain_gathers(p):
      pltpu.make_async_copy(z8_h.at[pl.ds(0, ch)], rows.at[p], gsem[p]).wait()

    def fire_scatters(p):
      def fs(j, carry):
        pltpu.async_copy(rows.at[p].at[pl.ds(j * _IDXW, _IDXW)],
                         acc_s.at[dstb.at[p].at[j]], ssem[p], add=True)
        if with_count:
          pltpu.async_copy(onesv, cnt_s.at[dstb.at[p].at[j]], ssem[p],
                           add=True)
        return carry
      lax.fori_loop(0, kr, fs, 0)

    def drain_scatters(p):
      pltpu.make_async_copy(z8_h.at[pl.ds(0, ch)], rows.at[p], ssem[p]).wait()
      if with_count:
        pltpu.make_async_copy(z1_h.at[pl.ds(0, ch)], ewb.at[p],
                              ssem[p]).wait()

    def multiply(p):
      rows_p = rows.at[p]
      ew_p = ewb.at[p]
      def mb(i, ridx):
        v = plsc.load_gather(rows_p, [ridx, col])
        w = plsc.load_gather(ew_p, [ridx])
        plsc.store_scatter(rows_p, [ridx, col], v * w)
        return ridx + 2
      lax.fori_loop(0, ch * dh // _LANES, mb, step, unroll=8)

    load_idx(0, 0)
    load_idx(1, 1)
    drain_idx(0)
    fire_gathers(0)
    drain_idx(1)
    fire_gathers(1)
    drain_gathers(0)
    multiply(0)
    fire_scatters(0)

    def steady(c, p):
      q = 1 - p
      drain_scatters(q)
      @pl.when(c + 1 < n_chunks)
      def _():
        load_idx(c + 1, q)
      drain_gathers(p)
      @pl.when(c + 1 < n_chunks)
      def _():
        drain_idx(q)
        fire_gathers(q)
      multiply(p)
      fire_scatters(p)

    def pair_body(g, carry):
      steady(2 * g + 1, 1)
      steady(2 * g + 2, 0)
      return carry
    lax.fori_loop(0, (n_chunks - 1) // 2, pair_body, 0)
    if n_chunks % 2 == 0:
      steady(n_chunks - 1, 1)
    drain_scatters((n_chunks - 1) & 1)

    if rem_rows:
      @pl.when(wid < rem_rows)
      def _():
        row = start_row + base_rows
        pltpu.sync_copy(src_h.at[pl.ds(row, 1)], srcb.at[0].at[pl.ds(0, 1)])
        pltpu.sync_copy(dst_h.at[pl.ds(row, 1)], dstb.at[0].at[pl.ds(0, 1)])
        pltpu.sync_copy(ew_h.at[pl.ds(row * _IDXW, _IDXW)],
                        ewb.at[0].at[pl.ds(0, _IDXW)])
        pltpu.async_copy(table.at[srcb.at[0].at[0]],
                         rows.at[0].at[pl.ds(0, _IDXW)], gsem[0]).wait()
        rows_t = rows.at[0]
        ew_t = ewb.at[0]
        def mb_t(i, ridx):
          v = plsc.load_gather(rows_t, [ridx, col])
          w = plsc.load_gather(ew_t, [ridx])
          plsc.store_scatter(rows_t, [ridx, col], v * w)
          return ridx + 2
        lax.fori_loop(0, _IDXW * dh // _LANES, mb_t, step)
        pltpu.sync_copy(rows.at[0].at[pl.ds(0, _IDXW)],
                        acc_s.at[dstb.at[0].at[0]], add=True)
        if with_count:
          pltpu.sync_copy(onesv, cnt_s.at[dstb.at[0].at[0]], add=True)

    plsc.subcore_barrier()

    copy_span(lambda o, r: acc_s.at[pl.ds(o, r)],
              lambda o, r: acc_o.at[c].at[pl.ds(o, r)])
    if with_count:
      @pl.when(c == 0)
      def _():
        copy_span(lambda o, r: cnt_s.at[pl.ds(o, r)],
                  lambda o, r: cnt0_o.at[pl.ds(o, r)])

      @pl.when(c == 1)
      def _():
        copy_span(lambda o, r: cnt_s.at[pl.ds(o, r)],
                  lambda o, r: cnt1_o.at[pl.ds(o, r)])

  return pl.kernel(
      body, out_type=out_type, mesh=mesh, scratch_types=scratch,
      compiler_params=pltpu.CompilerParams(
          needs_layout_passes=False, use_tc_tiling_on_sc=False))


def _tc_pre(x256, wbd_l, wbd_r):
  def body(x_ref, wl_ref, wr_ref, y_ref, xr_ref):
    xv = x_ref[...]
    y_ref[...] = jnp.dot(xv, wl_ref[...], preferred_element_type=jnp.float32)
    xr_ref[...] = jnp.dot(xv, wr_ref[...], preferred_element_type=jnp.float32)

  r, k = x256.shape
  blk = _TC_BLK
  grid = -(-r // blk)
  row_in = pl.BlockSpec((blk, k), lambda i: (i, 0))
  row = pl.BlockSpec((blk, 128), lambda i: (i, 0))
  wfull = pl.BlockSpec((k, 128), lambda i: (0, 0))
  return pl.pallas_call(
      body,
      grid=(grid,),
      in_specs=[row_in, wfull, wfull],
      out_specs=[row, row],
      out_shape=[jax.ShapeDtypeStruct((r, 128), jnp.float32)] * 2,
  )(x256, wbd_l, wbd_r)


def _tc_mid(accg, ca, cb, xrg, b128, wbd_l, wbd_r):
  def body(acc_ref, ca_ref, cb_ref, xr_ref, b_ref, wl_ref, wr_ref,
           y2_ref, hr_ref, ct_ref):
    ctot = ca_ref[...] + cb_ref[...]
    h = (acc_ref[0] + acc_ref[1]) / jnp.maximum(ctot, 1.0)
    h = jnp.maximum(h + b_ref[...] + xr_ref[...], 0.0)
    y2_ref[...] = jnp.dot(h, wl_ref[...], preferred_element_type=jnp.float32)
    hr_ref[...] = jnp.dot(h, wr_ref[...], preferred_element_type=jnp.float32)
    ct_ref[...] = ctot

  r = accg.shape[1]
  blk = _TC_BLK
  grid = -(-r // blk)
  rowp = pl.BlockSpec((_NCORES, blk, 128), lambda i: (0, i, 0))
  row = pl.BlockSpec((blk, 128), lambda i: (i, 0))
  bias = pl.BlockSpec((1, 128), lambda i: (0, 0))
  wfull = pl.BlockSpec((128, 128), lambda i: (0, 0))
  return pl.pallas_call(
      body,
      grid=(grid,),
      in_specs=[rowp, row, row, row, bias, wfull, wfull],
      out_specs=[row, row, row],
      out_shape=[jax.ShapeDtypeStruct((r, 128), jnp.float32)] * 3,
  )(accg, ca, cb, xrg, b128, wbd_l, wbd_r)


def _tc_post(accg, ctg, hrg, b128, ksum, p4, p2, p1):
  def body(acc_ref, ct_ref, hr_ref, b_ref, ks_ref, p4_ref, p2_ref, p1_ref,
           out_ref):
    z = (acc_ref[0] + acc_ref[1]) / jnp.maximum(ct_ref[...], 1.0)
    z = z + b_ref[...] + hr_ref[...]
    m = jnp.maximum(z, jnp.dot(z, p4_ref[...],
                               preferred_element_type=jnp.float32))
    m = jnp.maximum(m, jnp.dot(m, p2_ref[...],
                               preferred_element_type=jnp.float32))
    m = jnp.maximum(m, jnp.dot(m, p1_ref[...],
                               preferred_element_type=jnp.float32))
    e = jnp.exp(z - m)
    gs = jnp.dot(e, ks_ref[...], preferred_element_type=jnp.float32)
    out_ref[...] = z - m - jnp.log(gs)

  r = accg.shape[1]
  blk = _TC_BLK
  grid = -(-r // blk)
  rowp = pl.BlockSpec((_NCORES, blk, 128), lambda i: (0, i, 0))
  row = pl.BlockSpec((blk, 128), lambda i: (i, 0))
  bias = pl.BlockSpec((1, 128), lambda i: (0, 0))
  wfull = pl.BlockSpec((128, 128), lambda i: (0, 0))
  return pl.pallas_call(
      body,
      grid=(grid,),
      in_specs=[rowp, row, row, bias, wfull, wfull, wfull, wfull],
      out_specs=row,
      out_shape=jax.ShapeDtypeStruct((r, 128), jnp.float32),
  )(accg, ctg, hrg, b128, ksum, p4, p2, p1)


def kernel(x, edge_index, edge_weight, W1_l, b1, W1_r, W3_l, b3, W3_r):
  n, din = x.shape
  dh = W1_l.shape[1]
  e = edge_weight.shape[0]
  pack = 128 // dh
  r8 = n * dh // 128

  rows_total = -(-e // _IDXW)
  pad = rows_total * _IDXW - e
  if pad:
    edge_index = jnp.concatenate(
        [edge_index,
         jnp.stack([jnp.zeros((pad,), jnp.int32),
                    jnp.full((pad,), n, jnp.int32)])], axis=1)
    edge_weight = jnp.concatenate(
        [edge_weight, jnp.zeros((pad,), jnp.float32)])
  edge3d = edge_index.reshape(2, rows_total, _IDXW)
  ew = edge_weight
  z8 = jnp.zeros((n, dh), jnp.float32)
  z1 = jnp.zeros((n,), jnp.float32)

  eye = jnp.eye(pack, dtype=jnp.float32)
  w1bd_l = jnp.kron(eye, W1_l)
  w1bd_r = jnp.kron(eye, W1_r)
  w3bd_l = jnp.kron(eye, W3_l)
  w3bd_r = jnp.kron(eye, W3_r)
  ksum = jnp.kron(eye, jnp.ones((dh, dh), jnp.float32))
  lane = jnp.arange(128)
  g, p = lane // dh, lane % dh
  def perm(k):
    m = jnp.zeros((128, 128), jnp.float32)
    return m.at[g * dh + (p + k) % dh, lane].set(1.0)
  p4, p2, p1 = perm(4), perm(2), perm(1)
  b1_128 = jnp.tile(b1, pack).reshape(1, 128)
  b3_128 = jnp.tile(b3, pack).reshape(1, 128)

  sc_count = _make_sc_pass(n, dh, rows_total, with_count=True)
  sc_plain = _make_sc_pass(n, dh, rows_total, with_count=False)

  y1g, xrg = _tc_pre(x.reshape(n // pack, pack * din), w1bd_l, w1bd_r)
  acc1, cnt_a, cnt_b = sc_count(y1g.reshape(n, dh), edge3d, ew, z8, z1)
  ca = jnp.broadcast_to(cnt_a[:, None], (n, dh)).reshape(r8, 128)
  cb = jnp.broadcast_to(cnt_b[:, None], (n, dh)).reshape(r8, 128)
  y2g, hrg, ctg = _tc_mid(acc1.reshape(_NCORES, r8, 128), ca, cb, xrg,
                          b1_128, w3bd_l, w3bd_r)

  (acc2,) = sc_plain(y2g.reshape(n, dh), edge3d, ew, z8)
  out = _tc_post(acc2.reshape(_NCORES, r8, 128), ctg, hrg, b3_128,
                 ksum, p4, p2, p1)
  return out.reshape(n, dh)
